# Initial kernel scaffold; baseline (speedup 1.0000x reference)
import jax
import jax.numpy as jnp
from jax.experimental import pallas as pl


def kernel(x, table):
    raise NotImplementedError("write your pallas kernel here")


# speedup vs baseline: 2.4239x; 2.4239x over previous
import functools
import math

import jax
import jax.numpy as jnp
from jax import lax
from jax.experimental import pallas as pl
from jax.experimental.pallas import tpu as pltpu
from jax.experimental.pallas import tpu_sc as plsc

D = 128
SCALE = math.sqrt(128.0)
NW = 32
K = 128


@functools.lru_cache(maxsize=None)
def _make_kernel(n_rows: int):
    per_w = n_rows // NW
    n_chunks = per_w // K
    mesh = plsc.VectorSubcoreMesh(core_axis_name="c", subcore_axis_name="s")

    @functools.partial(
        pl.kernel,
        out_type=jax.ShapeDtypeStruct((n_rows, D), jnp.float32),
        mesh=mesh,
        scratch_types=[
            pltpu.VMEM((n_chunks, K), jnp.int32),
            pltpu.VMEM((K, D), jnp.float32),
            pltpu.SemaphoreType.DMA,
        ],
    )
    def emb(idx_hbm, table_hbm, out_hbm, idx_v, buf, sem):
        wid = lax.axis_index("s") * 2 + lax.axis_index("c")
        pltpu.sync_copy(idx_hbm.at[wid], idx_v)
        base = wid * per_w

        def chunk(j, carry):
            pltpu.async_copy(table_hbm.at[idx_v.at[j]], buf, sem).wait()

            def row(i, c2):
                for c in range(D // 16):
                    s = pl.ds(c * 16, 16)
                    buf[i, s] = buf[i, s] * SCALE
                return c2

            lax.fori_loop(0, K, row, 0)
            pltpu.sync_copy(buf, out_hbm.at[pl.ds(base + j * K, K)])
            return carry

        lax.fori_loop(0, n_chunks, chunk, 0)

    return emb


def kernel(x, table):
    n_rows = x.shape[0] * x.shape[1]
    idx = x.reshape(NW, n_rows // NW // K, K).astype(jnp.int32)
    out = _make_kernel(n_rows)(idx, table)
    return out.reshape(x.shape[0], x.shape[1], D)


# speedup vs baseline: 2.9436x; 1.2144x over previous
import functools
import math

import jax
import jax.numpy as jnp
from jax import lax
from jax.experimental import pallas as pl
from jax.experimental.pallas import tpu as pltpu
from jax.experimental.pallas import tpu_sc as plsc

D = 128
SCALE = math.sqrt(128.0)
NW = 32
K = 128


@functools.lru_cache(maxsize=None)
def _make_kernel(n_rows: int):
    per_w = n_rows // NW
    n_chunks = per_w // K
    assert n_chunks >= 4 and n_chunks % 2 == 0
    mesh = plsc.VectorSubcoreMesh(core_axis_name="c", subcore_axis_name="s")

    @functools.partial(
        pl.kernel,
        out_type=jax.ShapeDtypeStruct((n_rows, D), jnp.float32),
        mesh=mesh,
        scratch_types=[
            pltpu.VMEM((n_chunks, K), jnp.int32),
            pltpu.VMEM((2, K, D), jnp.float32),
            pltpu.VMEM((2, K, D), jnp.float32),
            pltpu.SemaphoreType.DMA,
            pltpu.SemaphoreType.DMA,
            pltpu.SemaphoreType.DMA,
            pltpu.SemaphoreType.DMA,
        ],
    )
    def emb(idx_hbm, table_hbm, out_hbm, idx_v, gbuf, sbuf,
            gsem0, gsem1, ssem0, ssem1):
        wid = lax.axis_index("s") * 2 + lax.axis_index("c")
        pltpu.sync_copy(idx_hbm.at[wid], idx_v)
        base = wid * per_w
        gsems = (gsem0, gsem1)
        ssems = (ssem0, ssem1)

        def fire_gather(b, j):
            pltpu.async_copy(table_hbm.at[idx_v.at[j]], gbuf.at[b], gsems[b])

        def wait_gather(b):
            pltpu.make_async_copy(
                table_hbm.at[pl.ds(0, K)], gbuf.at[b], gsems[b]).wait()

        def fire_store(b, j):
            pltpu.async_copy(
                sbuf.at[b], out_hbm.at[pl.ds(base + j * K, K)], ssems[b])

        def wait_store(b):
            pltpu.make_async_copy(
                sbuf.at[b], out_hbm.at[pl.ds(0, K)], ssems[b]).wait()

        def scale(b):
            def row(i, c):
                for c8 in range(D // 16):
                    s = pl.ds(c8 * 16, 16)
                    sbuf[b, i, s] = gbuf[b, i, s] * SCALE
                return c
            lax.fori_loop(0, K, row, 0)

        fire_gather(0, 0)
        fire_gather(1, 1)
        for j in range(2):
            b = j % 2
            wait_gather(b)
            scale(b)
            fire_store(b, j)
            fire_gather(b, j + 2)

        def group(j2, c):
            for b in range(2):
                j = 2 * j2 + b
                wait_gather(b)
                wait_store(b)
                scale(b)
                fire_store(b, j)
                fire_gather(b, j + 2)
            return c

        lax.fori_loop(1, n_chunks // 2 - 1, group, 0)

        for j in range(n_chunks - 2, n_chunks):
            b = j % 2
            wait_gather(b)
            wait_store(b)
            scale(b)
            fire_store(b, j)
        wait_store(0)
        wait_store(1)

    return emb


def kernel(x, table):
    n_rows = x.shape[0] * x.shape[1]
    idx = x.reshape(NW, n_rows // NW // K, K).astype(jnp.int32)
    out = _make_kernel(n_rows)(idx, table)
    return out.reshape(x.shape[0], x.shape[1], D)


# speedup vs baseline: 4.0819x; 1.3867x over previous
import functools
import math

import jax
import jax.numpy as jnp
from jax import lax
from jax.experimental import pallas as pl
from jax.experimental.pallas import tpu as pltpu
from jax.experimental.pallas import tpu_sc as plsc

D = 128
SCALE = math.sqrt(128.0)
NW = 32


@functools.lru_cache(maxsize=None)
def _make_kernel(B: int, L: int):
    rows_per_w = B // NW
    lp = (L + 7) // 8 * 8
    assert rows_per_w >= 4 and rows_per_w % 2 == 0 and L <= 128
    mesh = plsc.VectorSubcoreMesh(core_axis_name="c", subcore_axis_name="s")

    @functools.partial(
        pl.kernel,
        out_type=jax.ShapeDtypeStruct((B, lp, D), jnp.float32),
        mesh=mesh,
        compiler_params=pltpu.CompilerParams(use_tc_tiling_on_sc=False),
        scratch_types=[
            pltpu.VMEM((rows_per_w, L), jnp.int32),
            pltpu.VMEM((2, L, D), jnp.float32),
            pltpu.VMEM((2, lp, D), jnp.float32),
            pltpu.SemaphoreType.DMA,
            pltpu.SemaphoreType.DMA,
            pltpu.SemaphoreType.DMA,
            pltpu.SemaphoreType.DMA,
        ],
    )
    def emb(idx_hbm, table_hbm, out_hbm, idx_v, gbuf, sbuf,
            gsem0, gsem1, ssem0, ssem1):
        wid = lax.axis_index("s") * 2 + lax.axis_index("c")
        pltpu.sync_copy(idx_hbm.at[wid], idx_v)
        base = wid * rows_per_w
        gsems = (gsem0, gsem1)
        ssems = (ssem0, ssem1)

        def fire_gather(b, j):
            pltpu.async_copy(table_hbm.at[idx_v.at[j]], gbuf.at[b], gsems[b])

        def wait_gather(b):
            pltpu.make_async_copy(
                table_hbm.at[pl.ds(0, L)], gbuf.at[b], gsems[b]).wait()

        def fire_store(b, j):
            pltpu.async_copy(sbuf.at[b], out_hbm.at[base + j], ssems[b])

        def wait_store(b):
            pltpu.make_async_copy(
                sbuf.at[b], out_hbm.at[0], ssems[b]).wait()

        def scale(b):
            def row(i, c):
                for c8 in range(D // 16):
                    s = pl.ds(c8 * 16, 16)
                    sbuf[b, i, s] = gbuf[b, i, s] * SCALE
                return c
            lax.fori_loop(0, L, row, 0)

        fire_gather(0, 0)
        fire_gather(1, 1)
        for j in range(2):
            b = j % 2
            wait_gather(b)
            scale(b)
            fire_store(b, j)
            fire_gather(b, j + 2)

        def group(j2, c):
            for b in range(2):
                j = 2 * j2 + b
                wait_gather(b)
                wait_store(b)
                scale(b)
                fire_store(b, j)
                fire_gather(b, j + 2)
            return c

        lax.fori_loop(1, rows_per_w // 2 - 1, group, 0)

        for j in range(rows_per_w - 2, rows_per_w):
            b = j % 2
            wait_gather(b)
            wait_store(b)
            scale(b)
            fire_store(b, j)
        wait_store(0)
        wait_store(1)

    return emb


def kernel(x, table):
    B, L = x.shape
    idx = x.reshape(NW, B // NW, L).astype(jnp.int32)
    out = _make_kernel(B, L)(idx, table)
    return out[:, :L, :]
